# Initial kernel scaffold; baseline (speedup 1.0000x reference)
import jax
import jax.numpy as jnp
from jax.experimental import pallas as pl


def kernel(x, token_embedding, pos_embedding):
    raise NotImplementedError("write your pallas kernel here")


# speedup vs baseline: 3.9600x; 3.9600x over previous
import functools

import jax
import jax.numpy as jnp
from jax import lax
from jax.experimental import pallas as pl
from jax.experimental.pallas import tpu as pltpu
import jax.experimental.pallas.tpu_sc as plsc

_NC = 2
_NS = 16
_LANES = 16


def kernel(x, token_embedding, pos_embedding):
    B, S = x.shape
    V, D = token_embedding.shape
    NW = _NC * _NS
    rows_per_w = B // NW
    C = 2
    SC_ = S // C

    x3 = x.astype(jnp.int32).reshape(B, C, SC_)

    mesh = plsc.VectorSubcoreMesh(core_axis_name="c", subcore_axis_name="s")

    @functools.partial(
        pl.kernel,
        out_type=jax.ShapeDtypeStruct((B, S, D), jnp.float32),
        mesh=mesh,
        scratch_types=[
            pltpu.VMEM((C, SC_), jnp.int32),
            pltpu.VMEM((S, D), jnp.float32),
            pltpu.VMEM((S, D), jnp.float32),
            pltpu.SemaphoreType.DMA,
        ],
    )
    def emb(x_hbm, tok_hbm, pos_hbm, out_hbm, idx_v, rows_v, pos_v, sem):
        wid = lax.axis_index("s") * _NC + lax.axis_index("c")
        pltpu.sync_copy(pos_hbm, pos_v)

        def row_body(r, carry):
            row = wid * rows_per_w + r
            pltpu.sync_copy(x_hbm.at[row], idx_v)
            for c in range(C):
                pltpu.async_copy(
                    tok_hbm.at[idx_v.at[c]], rows_v.at[pl.ds(c * SC_, SC_)], sem
                )
            for c in range(C):
                pltpu.make_async_copy(
                    tok_hbm.at[idx_v.at[c]], rows_v.at[pl.ds(c * SC_, SC_)], sem
                ).wait()

            def add_j(j, c2):
                for i in range(D // _LANES):
                    sl = pl.ds(i * _LANES, _LANES)
                    plsc.addupdate(rows_v.at[j, sl], pos_v[j, sl])
                return c2

            lax.fori_loop(0, S, add_j, 0)
            pltpu.sync_copy(rows_v, out_hbm.at[row])
            return carry

        lax.fori_loop(0, rows_per_w, row_body, 0)

    return emb(x3, token_embedding, pos_embedding)


# speedup vs baseline: 7.0142x; 1.7712x over previous
import functools

import jax
import jax.numpy as jnp
from jax import lax
from jax.experimental import pallas as pl
from jax.experimental.pallas import tpu as pltpu
import jax.experimental.pallas.tpu_sc as plsc

_NC = 2
_NS = 16
_LANES = 16
_NBUF = 5
_LOOK = 3


def kernel(x, token_embedding, pos_embedding):
    B, S = x.shape
    V, D = token_embedding.shape
    NW = _NC * _NS
    rows_per_w = B // NW
    C = _NBUF
    SC_ = S // C
    n_outer = rows_per_w

    x3 = x.astype(jnp.int32).reshape(B, C, SC_)

    mesh = plsc.VectorSubcoreMesh(core_axis_name="c", subcore_axis_name="s")

    @functools.partial(
        pl.kernel,
        out_type=jax.ShapeDtypeStruct((B, S, D), jnp.float32),
        mesh=mesh,
        scratch_types=[
            pltpu.VMEM((rows_per_w, C, SC_), jnp.int32),
            pltpu.VMEM((_NBUF, SC_, D), jnp.float32),
            pltpu.VMEM((S, D), jnp.float32),
            [pltpu.SemaphoreType.DMA] * _NBUF,
            [pltpu.SemaphoreType.DMA] * _NBUF,
        ],
    )
    def emb(x_hbm, tok_hbm, pos_hbm, out_hbm, idx_all, rows_v, pos_v, gsem, osem):
        wid = lax.axis_index("s") * _NC + lax.axis_index("c")
        base_row = wid * rows_per_w
        pltpu.sync_copy(pos_hbm, pos_v)
        pltpu.sync_copy(x_hbm.at[pl.ds(base_row, rows_per_w)], idx_all)

        def gather_desc(lr, c):
            return pltpu.make_async_copy(
                tok_hbm.at[idx_all.at[lr, c]], rows_v.at[c], gsem[c]
            )

        def store_desc(lr, c):
            return pltpu.make_async_copy(
                rows_v.at[c], out_hbm.at[base_row + lr, pl.ds(c * SC_, SC_)], osem[c]
            )

        def posadd(c):
            @plsc.parallel_loop(0, SC_, unroll=2)
            def _(j):
                for i in range(D // _LANES):
                    sl = pl.ds(i * _LANES, _LANES)
                    plsc.addupdate(rows_v.at[c, j, sl], pos_v[c * SC_ + j, sl])

        def unit_step(g, k, *, drain, prefetch):
            if drain:
                store_desc(g + (k - 2) // _NBUF, (k - 2) % _NBUF).wait()
            if prefetch:
                gather_desc(g + (k + _LOOK) // _NBUF, (k + _LOOK) % _NBUF).start()
            gather_desc(g, k).wait()
            posadd(k)
            store_desc(g, k).start()

        for k in range(_LOOK):
            gather_desc(0, k).start()

        for k in range(_NBUF):
            unit_step(0, k, drain=(k >= _NBUF - _LOOK), prefetch=True)

        def outer(g, carry):
            for k in range(_NBUF):
                unit_step(g, k, drain=True, prefetch=True)
            return carry

        lax.fori_loop(1, n_outer - 1, outer, 0)

        for k in range(_NBUF):
            unit_step(n_outer - 1, k, drain=True, prefetch=(k < _NBUF - _LOOK))

        for k in range(_NBUF - 2, _NBUF):
            store_desc(n_outer - 1, k).wait()

    return emb(x3, token_embedding, pos_embedding)
